# baseline (speedup 1.0000x reference)
import functools

import jax
import jax.numpy as jnp
from jax import lax
from jax.experimental import pallas as pl
from jax.experimental.pallas import tpu as pltpu
from jax.experimental.pallas import tpu_sc as plsc

BATCH = 1024
SEQ = 20
EMB_DIM = 128
NUM_CORES = 2
NUM_SUBCORES = 16
NUM_WORKERS = NUM_CORES * NUM_SUBCORES
TOTAL = BATCH * SEQ
PER_WORKER = TOTAL // NUM_WORKERS
CHUNK = 128
NUM_CHUNKS = PER_WORKER // CHUNK

_mesh = plsc.VectorSubcoreMesh(core_axis_name="c", subcore_axis_name="s",
                               num_cores=NUM_CORES,
                               num_subcores=NUM_SUBCORES)


@functools.partial(
    pl.kernel,
    out_type=jax.ShapeDtypeStruct((TOTAL, EMB_DIM), jnp.float32),
    mesh=_mesh,
    scratch_types=[
        pltpu.VMEM((PER_WORKER,), jnp.int32),
        pltpu.VMEM((PER_WORKER, EMB_DIM), jnp.float32),
        pltpu.VMEM_SHARED((3882, EMB_DIM), jnp.float32),
        pltpu.SemaphoreType.DMA,
        pltpu.SemaphoreType.DMA,
        pltpu.SemaphoreType.DMA,
    ],
    compiler_params=pltpu.CompilerParams(use_tc_tiling_on_sc=True),
)
def _gather_kernel(idx_hbm, table_hbm, out_hbm, idx_v, rows_v, table_sh,
                   gsem, wsem, tsem):
    sid = lax.axis_index("s")
    wid = sid * NUM_CORES + lax.axis_index("c")
    base = wid * PER_WORKER
    pltpu.sync_copy(idx_hbm.at[pl.ds(base, PER_WORKER)], idx_v)
    rstart = sid * 248

    @pl.when(sid < 15)
    def _():
        pltpu.async_copy(table_hbm.at[pl.ds(rstart, 248)],
                         table_sh.at[pl.ds(rstart, 248)], tsem)

    @pl.when(sid == 15)
    def _():
        pltpu.async_copy(table_hbm.at[pl.ds(3720, 162)],
                         table_sh.at[pl.ds(3720, 162)], tsem)
    N_HBM = 2
    gathers = [
        pltpu.async_copy(table_hbm.at[idx_v.at[pl.ds(j * CHUNK, CHUNK)]],
                         rows_v.at[pl.ds(j * CHUNK, CHUNK)], gsem)
        for j in range(N_HBM)
    ]

    @pl.when(sid < 15)
    def _():
        pltpu.make_async_copy(table_hbm.at[pl.ds(rstart, 248)],
                              table_sh.at[pl.ds(rstart, 248)], tsem).wait()

    @pl.when(sid == 15)
    def _():
        pltpu.make_async_copy(table_hbm.at[pl.ds(3720, 162)],
                              table_sh.at[pl.ds(3720, 162)], tsem).wait()
    plsc.subcore_barrier()
    gathers += [
        pltpu.async_copy(table_sh.at[idx_v.at[pl.ds(j * CHUNK, CHUNK)]],
                         rows_v.at[pl.ds(j * CHUNK, CHUNK)], gsem)
        for j in range(N_HBM, NUM_CHUNKS)
    ]
    writes = []
    for j in range(NUM_CHUNKS):
        gathers[j].wait()
        writes.append(
            pltpu.async_copy(rows_v.at[pl.ds(j * CHUNK, CHUNK)],
                             out_hbm.at[pl.ds(base + j * CHUNK, CHUNK)],
                             wsem))
    for w in writes:
        w.wait()


def kernel(inputs, embeddingDict):
    idx = inputs.T.reshape(TOTAL)
    out = _gather_kernel(idx, embeddingDict)
    return out.reshape(SEQ, BATCH, EMB_DIM).transpose(1, 0, 2)


# speedup vs baseline: 1.0378x; 1.0378x over previous
import functools

import jax
import jax.numpy as jnp
from jax import lax
from jax.experimental import pallas as pl
from jax.experimental.pallas import tpu as pltpu
from jax.experimental.pallas import tpu_sc as plsc

BATCH = 1024
SEQ = 20
EMB_DIM = 128
NUM_CORES = 2
NUM_SUBCORES = 16
NUM_WORKERS = NUM_CORES * NUM_SUBCORES
TOTAL = BATCH * SEQ
PER_WORKER = TOTAL // NUM_WORKERS
CHUNK = 128
NUM_CHUNKS = PER_WORKER // CHUNK

_mesh = plsc.VectorSubcoreMesh(core_axis_name="c", subcore_axis_name="s",
                               num_cores=NUM_CORES,
                               num_subcores=NUM_SUBCORES)


@functools.partial(
    pl.kernel,
    out_type=jax.ShapeDtypeStruct((TOTAL, EMB_DIM), jnp.float32),
    mesh=_mesh,
    scratch_types=[
        pltpu.VMEM((PER_WORKER,), jnp.int32),
        pltpu.VMEM((PER_WORKER, EMB_DIM), jnp.float32),
        pltpu.VMEM_SHARED((3882, EMB_DIM), jnp.float32),
        pltpu.SemaphoreType.DMA,
        pltpu.SemaphoreType.DMA,
        pltpu.SemaphoreType.DMA,
    ],
    compiler_params=pltpu.CompilerParams(use_tc_tiling_on_sc=True),
)
def _gather_kernel(idx_hbm, table_hbm, out_hbm, idx_v, rows_v, table_sh,
                   gsem, wsem, tsem):
    sid = lax.axis_index("s")
    wid = sid * NUM_CORES + lax.axis_index("c")
    base = wid * PER_WORKER
    rstart = sid * 248

    @pl.when(sid < 15)
    def _():
        pltpu.async_copy(table_hbm.at[pl.ds(rstart, 248)],
                         table_sh.at[pl.ds(rstart, 248)], tsem)

    @pl.when(sid == 15)
    def _():
        pltpu.async_copy(table_hbm.at[pl.ds(3720, 162)],
                         table_sh.at[pl.ds(3720, 162)], tsem)
    pltpu.sync_copy(idx_hbm.at[pl.ds(base, PER_WORKER)], idx_v)
    N_HBM = 2
    gathers = [
        pltpu.async_copy(table_hbm.at[idx_v.at[pl.ds(j * CHUNK, CHUNK)]],
                         rows_v.at[pl.ds(j * CHUNK, CHUNK)], gsem)
        for j in range(N_HBM)
    ]

    @pl.when(sid < 15)
    def _():
        pltpu.make_async_copy(table_hbm.at[pl.ds(rstart, 248)],
                              table_sh.at[pl.ds(rstart, 248)], tsem).wait()

    @pl.when(sid == 15)
    def _():
        pltpu.make_async_copy(table_hbm.at[pl.ds(3720, 162)],
                              table_sh.at[pl.ds(3720, 162)], tsem).wait()
    plsc.subcore_barrier()
    gathers += [
        pltpu.async_copy(table_sh.at[idx_v.at[pl.ds(j * CHUNK, CHUNK)]],
                         rows_v.at[pl.ds(j * CHUNK, CHUNK)], gsem)
        for j in range(N_HBM, NUM_CHUNKS)
    ]
    writes = []
    for j in range(NUM_CHUNKS):
        gathers[j].wait()
        writes.append(
            pltpu.async_copy(rows_v.at[pl.ds(j * CHUNK, CHUNK)],
                             out_hbm.at[pl.ds(base + j * CHUNK, CHUNK)],
                             wsem))
    for w in writes:
        w.wait()


def kernel(inputs, embeddingDict):
    idx = inputs.T.reshape(TOTAL)
    out = _gather_kernel(idx, embeddingDict)
    return out.reshape(SEQ, BATCH, EMB_DIM).transpose(1, 0, 2)
